# Initial kernel scaffold; baseline (speedup 1.0000x reference)
import jax
import jax.numpy as jnp
from jax.experimental import pallas as pl


def kernel(xyz, params):
    raise NotImplementedError("write your pallas kernel here")


# speedup vs baseline: 12.5522x; 12.5522x over previous
import functools

import jax
import jax.numpy as jnp
from jax.experimental import pallas as pl


def _relu(x):
    return jnp.maximum(x, 0.0)


def _dot(a, b):
    return jax.lax.dot_general(a, b, (((1,), (0,)), ((), ())),
                               preferred_element_type=jnp.float32)


def _fold_bn(layer):
    W, b = layer["W"], layer["b"]
    if "gamma" in layer:
        g, beta = layer["gamma"], layer["beta"]
        W = W * g[None, :]
        b = b * g + beta
    return W, b.reshape(1, -1)


def _fps_kernel(xyz_ref, out_ref, *, npoint):
    x = xyz_ref[:, 0, :]
    y = xyz_ref[:, 1, :]
    z = xyz_ref[:, 2, :]
    B, N = x.shape
    iota_n = jax.lax.broadcasted_iota(jnp.int32, (B, N), 1)
    iota_p = jax.lax.broadcasted_iota(jnp.int32, (B, npoint), 1)

    def body(i, st):
        dist, far, ax, ay, az = st
        oh = (iota_n == far).astype(jnp.float32)
        cx = jnp.sum(oh * x, axis=1, keepdims=True)
        cy = jnp.sum(oh * y, axis=1, keepdims=True)
        cz = jnp.sum(oh * z, axis=1, keepdims=True)
        d = (x - cx) ** 2 + (y - cy) ** 2 + (z - cz) ** 2
        dist = jnp.minimum(dist, d)
        mx = jnp.max(dist, axis=1, keepdims=True)
        far = jnp.min(jnp.where(dist == mx, iota_n, N), axis=1, keepdims=True)
        sel = iota_p == i
        ax = jnp.where(sel, cx, ax)
        ay = jnp.where(sel, cy, ay)
        az = jnp.where(sel, cz, az)
        return dist, far, ax, ay, az

    init = (jnp.full((B, N), 1e10, jnp.float32),
            jnp.zeros((B, 1), jnp.int32),
            jnp.zeros((B, npoint), jnp.float32),
            jnp.zeros((B, npoint), jnp.float32),
            jnp.zeros((B, npoint), jnp.float32))
    _, _, ax, ay, az = jax.lax.fori_loop(0, npoint, body, init)
    out_ref[:, 0, :] = ax
    out_ref[:, 1, :] = ay
    out_ref[:, 2, :] = az


def _fps_call(xyz_cols, npoint):
    B = xyz_cols.shape[0]
    return pl.pallas_call(
        functools.partial(_fps_kernel, npoint=npoint),
        out_shape=jax.ShapeDtypeStruct((B, 3, npoint), jnp.float32),
    )(xyz_cols)


def _sa_kernel(tbl_ref, xyzc_ref, new_ref, *w_refs, r2, K, n_mlp):
    out_ref = w_refs[-1]
    w_refs = w_refs[:-1]
    ws = [w_refs[i][...] for i in range(len(w_refs))]
    mlp = [(ws[2 * i], ws[2 * i + 1]) for i in range(n_mlp)]
    o = 2 * n_mlp
    lfa1W, lfa1b = ws[o], ws[o + 1]
    attW, attb = ws[o + 2], ws[o + 3]
    lfa2W, lfa2b = ws[o + 4], ws[o + 5]

    tbl = tbl_ref[0]
    xc = xyzc_ref[0]
    new = new_ref[0]
    N, D = tbl.shape
    Ts = new.shape[0]

    sqx = jnp.sum(xc * xc, axis=0, keepdims=True)
    sqn = jnp.sum(new * new, axis=1, keepdims=True)
    d2 = sqn + sqx - 2.0 * _dot(new, xc)
    iota = jax.lax.broadcasted_iota(jnp.int32, (Ts, N), 1)

    m = d2 <= r2
    cands = []
    for _ in range(K):
        c = jnp.min(jnp.where(m, iota, N), axis=1, keepdims=True)
        m = jnp.logical_and(m, iota != c)
        cands.append(c)
    first = cands[0]
    cands = [jnp.where(c == N, first, c) for c in cands]

    sub = jnp.concatenate(
        [new, jnp.zeros((Ts, D - 3), jnp.float32)], axis=1)

    hA = None
    gs = []
    aa = []
    for k in range(K):
        oh = (iota == cands[k]).astype(jnp.float32)
        f = _dot(oh, tbl) - sub
        h = f
        for (W, b) in mlp:
            h = _relu(_dot(h, W) + b)
        hA = h if hA is None else jnp.maximum(hA, h)
        g = _relu(_dot(f, lfa1W) + lfa1b)
        gs.append(g)
        aa.append(_dot(g, attW) + attb)

    mx = aa[0]
    for a in aa[1:]:
        mx = jnp.maximum(mx, a)
    es = [jnp.exp(a - mx) for a in aa]
    ssum = es[0]
    for e in es[1:]:
        ssum = ssum + e
    pooled = gs[0] * es[0]
    for g, e in zip(gs[1:], es[1:]):
        pooled = pooled + g * e
    pooled = pooled / ssum
    bB = _relu(_dot(pooled, lfa2W) + lfa2b)

    out_ref[0] = jnp.concatenate([hA, bB], axis=1)


def _sa_call(tbl, xyz_cols, new_rows, p, radius, K, Ts):
    B, N, D = tbl.shape
    S = new_rows.shape[1]
    mlp = [_fold_bn(L) for L in p["mlp"]]
    lfa1 = _fold_bn(p["lfa1"])
    att = (p["att"]["W"], p["att"]["b"].reshape(1, -1))
    lfa2 = _fold_bn(p["lfa2"])
    wargs = []
    for W, b in mlp + [lfa1, att, lfa2]:
        wargs += [W, b]
    Cout = mlp[-1][0].shape[1] + lfa2[0].shape[1]
    grid = (B, S // Ts)
    wspecs = [pl.BlockSpec(w.shape, lambda b_, t_: (0,) * w.ndim)
              for w in wargs]
    return pl.pallas_call(
        functools.partial(_sa_kernel, r2=radius * radius, K=K,
                          n_mlp=len(mlp)),
        grid=grid,
        in_specs=[
            pl.BlockSpec((1, N, D), lambda b_, t_: (b_, 0, 0)),
            pl.BlockSpec((1, 3, N), lambda b_, t_: (b_, 0, 0)),
            pl.BlockSpec((1, Ts, 3), lambda b_, t_: (b_, t_, 0)),
        ] + wspecs,
        out_specs=pl.BlockSpec((1, Ts, Cout), lambda b_, t_: (b_, t_, 0)),
        out_shape=jax.ShapeDtypeStruct((B, S, Cout), jnp.float32),
    )(tbl, xyz_cols, new_rows, *wargs)


def _fp_kernel(*refs, n_mlp, has_p1):
    if has_p1:
        x1_ref, x2c_ref, p2_ref, p1_ref = refs[:4]
        w_refs = refs[4:-1]
    else:
        x1_ref, x2c_ref, p2_ref = refs[:3]
        w_refs = refs[3:-1]
    out_ref = refs[-1]
    ws = [w_refs[i][...] for i in range(len(w_refs))]
    mlp = [(ws[2 * i], ws[2 * i + 1]) for i in range(n_mlp)]

    x1 = x1_ref[0]
    x2c = x2c_ref[0]
    p2 = p2_ref[0]
    Tn = x1.shape[0]
    S = x2c.shape[1]

    sq1 = jnp.sum(x1 * x1, axis=1, keepdims=True)
    sq2 = jnp.sum(x2c * x2c, axis=0, keepdims=True)
    d2 = sq1 + sq2 - 2.0 * _dot(x1, x2c)
    iota = jax.lax.broadcasted_iota(jnp.int32, (Tn, S), 1)

    d = d2
    iks = []
    vks = []
    for _ in range(3):
        mn = jnp.min(d, axis=1, keepdims=True)
        ck = jnp.min(jnp.where(d == mn, iota, S), axis=1, keepdims=True)
        iks.append(ck)
        vks.append(mn)
        d = jnp.where(iota == ck, 1e30, d)
    wk = [1.0 / (v + 1e-8) for v in vks]
    wsum = wk[0] + wk[1] + wk[2]
    Wm = jnp.zeros((Tn, S), jnp.float32)
    for k in range(3):
        Wm = Wm + jnp.where(iota == iks[k], wk[k] / wsum, 0.0)
    interp = _dot(Wm, p2)

    h = jnp.concatenate([p1_ref[0], interp], axis=1) if has_p1 else interp
    for (W, b) in mlp:
        h = _relu(_dot(h, W) + b)
    out_ref[0] = h


def _fp_call(x1_rows, x2_cols, p2_rows, p1_rows, p, Tn):
    B, N, _ = x1_rows.shape
    S = x2_cols.shape[2]
    C2 = p2_rows.shape[2]
    mlp = [_fold_bn(L) for L in p["mlp"]]
    wargs = []
    for W, b in mlp:
        wargs += [W, b]
    Cout = mlp[-1][0].shape[1]
    grid = (B, N // Tn)
    in_specs = [
        pl.BlockSpec((1, Tn, 3), lambda b_, t_: (b_, t_, 0)),
        pl.BlockSpec((1, 3, S), lambda b_, t_: (b_, 0, 0)),
        pl.BlockSpec((1, S, C2), lambda b_, t_: (b_, 0, 0)),
    ]
    args = [x1_rows, x2_cols, p2_rows]
    if p1_rows is not None:
        C1 = p1_rows.shape[2]
        in_specs.append(pl.BlockSpec((1, Tn, C1), lambda b_, t_: (b_, t_, 0)))
        args.append(p1_rows)
    in_specs += [pl.BlockSpec(w.shape, lambda b_, t_: (0,) * w.ndim)
                 for w in wargs]
    args += wargs
    return pl.pallas_call(
        functools.partial(_fp_kernel, n_mlp=len(mlp),
                          has_p1=p1_rows is not None),
        grid=grid,
        in_specs=in_specs,
        out_specs=pl.BlockSpec((1, Tn, Cout), lambda b_, t_: (b_, t_, 0)),
        out_shape=jax.ShapeDtypeStruct((B, N, Cout), jnp.float32),
    )(*args)


def _head_kernel(x_ref, w1_ref, b1_ref, w2_ref, b2_ref, out_ref):
    h = _relu(_dot(x_ref[...], w1_ref[...]) + b1_ref[...])
    y = _dot(h, w2_ref[...]) + b2_ref[...]
    mx = jnp.max(y, axis=1, keepdims=True)
    lse = jnp.log(jnp.sum(jnp.exp(y - mx), axis=1, keepdims=True)) + mx
    out_ref[...] = y - lse


def _head_call(rows, p1, p2, Tr):
    R, C = rows.shape
    W1, b1 = _fold_bn(p1)
    W2, b2 = p2["W"], p2["b"].reshape(1, -1)
    NC = W2.shape[1]
    grid = (R // Tr,)
    return pl.pallas_call(
        _head_kernel,
        grid=grid,
        in_specs=[
            pl.BlockSpec((Tr, C), lambda t_: (t_, 0)),
            pl.BlockSpec(W1.shape, lambda t_: (0, 0)),
            pl.BlockSpec(b1.shape, lambda t_: (0, 0)),
            pl.BlockSpec(W2.shape, lambda t_: (0, 0)),
            pl.BlockSpec(b2.shape, lambda t_: (0, 0)),
        ],
        out_specs=pl.BlockSpec((Tr, NC), lambda t_: (t_, 0)),
        out_shape=jax.ShapeDtypeStruct((R, NC), jnp.float32),
    )(rows, W1, b1, W2, b2)


_LEVELS = [
    (1024, 0.1, 256),
    (256, 0.2, 256),
    (64, 0.4, 64),
    (16, 0.8, 16),
]


def kernel(xyz, params):
    B = xyz.shape[0]
    xyz_cols = xyz[:, 0:3, :]
    xyz_rows = xyz_cols.transpose(0, 2, 1)
    pts_rows = xyz[:, 3:6, :].transpose(0, 2, 1)

    xs_cols = [xyz_cols]
    xs_rows = [xyz_rows]
    ps_rows = [pts_rows]
    for li, (npoint, radius, Ts) in enumerate(_LEVELS):
        p = params[f"ra{li + 1}"]
        new_cols = _fps_call(xs_cols[-1], npoint)
        new_rows = new_cols.transpose(0, 2, 1)
        tbl = jnp.concatenate([xs_rows[-1], ps_rows[-1]], axis=2)
        out = _sa_call(tbl, xs_cols[-1], new_rows, p, radius, 16, Ts)
        xs_cols.append(new_cols)
        xs_rows.append(new_rows)
        ps_rows.append(out)

    fp_cfg = [
        ("fp4", 3, 4, 64),
        ("fp3", 2, 3, 256),
        ("fp2", 1, 2, 512),
        ("fp1", 0, 1, 1024),
    ]
    cur = ps_rows[4]
    for name, i1, i2, Tn in fp_cfg:
        p = params[name]
        p1 = ps_rows[i1] if i1 > 0 else None
        cur = _fp_call(xs_rows[i1], xs_cols[i2], cur, p1, p, Tn)

    l0_rows = cur
    N = l0_rows.shape[1]
    rows = l0_rows.reshape(B * N, l0_rows.shape[2])
    x = _head_call(rows, params["head1"], params["head2"], 2048)
    x = x.reshape(B, N, x.shape[1])
    l0_out = l0_rows.transpose(0, 2, 1)
    return x, l0_out


# speedup vs baseline: 13.4107x; 1.0684x over previous
import functools

import jax
import jax.numpy as jnp
from jax.experimental import pallas as pl


def _relu(x):
    return jnp.maximum(x, 0.0)


def _dot(a, b):
    return jax.lax.dot_general(a, b, (((1,), (0,)), ((), ())),
                               preferred_element_type=jnp.float32)


def _fold_bn(layer):
    W, b = layer["W"], layer["b"]
    if "gamma" in layer:
        g, beta = layer["gamma"], layer["beta"]
        W = W * g[None, :]
        b = b * g + beta
    return W, b.reshape(1, -1)


def _fps_stage(x, y, z, npoint):
    B, N = x.shape
    iota_n = jax.lax.broadcasted_iota(jnp.int32, (B, N), 1)
    iota_p = jax.lax.broadcasted_iota(jnp.int32, (B, npoint), 1)

    def body(i, st):
        dist, far, ax, ay, az = st
        oh = (iota_n == far).astype(jnp.float32)
        cx = jnp.sum(oh * x, axis=1, keepdims=True)
        cy = jnp.sum(oh * y, axis=1, keepdims=True)
        cz = jnp.sum(oh * z, axis=1, keepdims=True)
        d = (x - cx) ** 2 + (y - cy) ** 2 + (z - cz) ** 2
        dist = jnp.minimum(dist, d)
        mx = jnp.max(dist, axis=1, keepdims=True)
        far = jnp.min(jnp.where(dist == mx, iota_n, N), axis=1, keepdims=True)
        sel = iota_p == i
        ax = jnp.where(sel, cx, ax)
        ay = jnp.where(sel, cy, ay)
        az = jnp.where(sel, cz, az)
        return dist, far, ax, ay, az

    init = (jnp.full((B, N), 1e10, jnp.float32),
            jnp.zeros((B, 1), jnp.int32),
            jnp.zeros((B, npoint), jnp.float32),
            jnp.zeros((B, npoint), jnp.float32),
            jnp.zeros((B, npoint), jnp.float32))
    _, _, ax, ay, az = jax.lax.fori_loop(0, npoint, body, init)
    return ax, ay, az


def _fps_kernel(xyz_ref, o1_ref, o2_ref, o3_ref, o4_ref, *, npoints):
    x = xyz_ref[:, 0, :]
    y = xyz_ref[:, 1, :]
    z = xyz_ref[:, 2, :]
    for npoint, o_ref in zip(npoints, (o1_ref, o2_ref, o3_ref, o4_ref)):
        x, y, z = _fps_stage(x, y, z, npoint)
        o_ref[:, 0, :] = x
        o_ref[:, 1, :] = y
        o_ref[:, 2, :] = z


def _fps_call(xyz_cols, npoints):
    B = xyz_cols.shape[0]
    return pl.pallas_call(
        functools.partial(_fps_kernel, npoints=npoints),
        out_shape=[jax.ShapeDtypeStruct((B, 3, n), jnp.float32)
                   for n in npoints],
    )(xyz_cols)


def _sa_kernel(tbl_ref, xyzc_ref, new_ref, tri_ref, *w_refs, r2, K, n_mlp):
    out_ref = w_refs[-1]
    w_refs = w_refs[:-1]
    ws = [w_refs[i][...] for i in range(len(w_refs))]
    mlp = [(ws[2 * i], ws[2 * i + 1]) for i in range(n_mlp)]
    o = 2 * n_mlp
    lfa1W, lfa1b = ws[o], ws[o + 1]
    attW, attb = ws[o + 2], ws[o + 3]
    lfa2W, lfa2b = ws[o + 4], ws[o + 5]

    tbl = tbl_ref[0]
    xc = xyzc_ref[0]
    new = new_ref[0]
    N, D = tbl.shape
    Ts = new.shape[0]

    sqx = jnp.sum(xc * xc, axis=0, keepdims=True)
    sqn = jnp.sum(new * new, axis=1, keepdims=True)
    d2 = sqn + sqx - 2.0 * _dot(new, xc)

    m = d2 <= r2
    rank = _dot(m.astype(jnp.bfloat16), tri_ref[...])
    cnt = rank[:, N - 1:N]
    rankm = jnp.where(m, rank, 0.0)

    sub = jnp.concatenate(
        [new, jnp.zeros((Ts, D - 3), jnp.float32)], axis=1)

    hA = None
    gs = []
    aa = []
    f0 = None
    for k in range(K):
        oh = (rankm == float(k + 1)).astype(jnp.float32)
        f = _dot(oh, tbl)
        if k == 0:
            f0 = f
        else:
            f = jnp.where(cnt >= float(k + 1), f, f0)
        f = f - sub
        h = f
        for (W, b) in mlp:
            h = _relu(_dot(h, W) + b)
        hA = h if hA is None else jnp.maximum(hA, h)
        g = _relu(_dot(f, lfa1W) + lfa1b)
        gs.append(g)
        aa.append(_dot(g, attW) + attb)

    mx = aa[0]
    for a in aa[1:]:
        mx = jnp.maximum(mx, a)
    es = [jnp.exp(a - mx) for a in aa]
    ssum = es[0]
    for e in es[1:]:
        ssum = ssum + e
    pooled = gs[0] * es[0]
    for g, e in zip(gs[1:], es[1:]):
        pooled = pooled + g * e
    pooled = pooled / ssum
    bB = _relu(_dot(pooled, lfa2W) + lfa2b)

    out_ref[0] = jnp.concatenate([hA, bB], axis=1)


def _sa_call(tbl, xyz_cols, new_rows, p, radius, K, Ts):
    B, N, D = tbl.shape
    S = new_rows.shape[1]
    mlp = [_fold_bn(L) for L in p["mlp"]]
    lfa1 = _fold_bn(p["lfa1"])
    att = (p["att"]["W"], p["att"]["b"].reshape(1, -1))
    lfa2 = _fold_bn(p["lfa2"])
    wargs = []
    for W, b in mlp + [lfa1, att, lfa2]:
        wargs += [W, b]
    Cout = mlp[-1][0].shape[1] + lfa2[0].shape[1]
    grid = (B, S // Ts)
    tri = jnp.triu(jnp.ones((N, N), jnp.bfloat16))
    wspecs = [pl.BlockSpec(w.shape, lambda b_, t_: (0,) * w.ndim)
              for w in wargs]
    return pl.pallas_call(
        functools.partial(_sa_kernel, r2=radius * radius, K=K,
                          n_mlp=len(mlp)),
        grid=grid,
        in_specs=[
            pl.BlockSpec((1, N, D), lambda b_, t_: (b_, 0, 0)),
            pl.BlockSpec((1, 3, N), lambda b_, t_: (b_, 0, 0)),
            pl.BlockSpec((1, Ts, 3), lambda b_, t_: (b_, t_, 0)),
            pl.BlockSpec((N, N), lambda b_, t_: (0, 0)),
        ] + wspecs,
        out_specs=pl.BlockSpec((1, Ts, Cout), lambda b_, t_: (b_, t_, 0)),
        out_shape=jax.ShapeDtypeStruct((B, S, Cout), jnp.float32),
    )(tbl, xyz_cols, new_rows, tri, *wargs)


def _fp_kernel(*refs, n_mlp, has_p1):
    if has_p1:
        x1_ref, x2c_ref, p2_ref, p1_ref = refs[:4]
        w_refs = refs[4:-1]
    else:
        x1_ref, x2c_ref, p2_ref = refs[:3]
        w_refs = refs[3:-1]
    out_ref = refs[-1]
    ws = [w_refs[i][...] for i in range(len(w_refs))]
    mlp = [(ws[2 * i], ws[2 * i + 1]) for i in range(n_mlp)]

    x1 = x1_ref[0]
    x2c = x2c_ref[0]
    p2 = p2_ref[0]
    Tn = x1.shape[0]
    S = x2c.shape[1]

    sq1 = jnp.sum(x1 * x1, axis=1, keepdims=True)
    sq2 = jnp.sum(x2c * x2c, axis=0, keepdims=True)
    d2 = sq1 + sq2 - 2.0 * _dot(x1, x2c)
    iota = jax.lax.broadcasted_iota(jnp.int32, (Tn, S), 1)

    d = d2
    iks = []
    vks = []
    for _ in range(3):
        mn = jnp.min(d, axis=1, keepdims=True)
        ck = jnp.min(jnp.where(d == mn, iota, S), axis=1, keepdims=True)
        iks.append(ck)
        vks.append(mn)
        d = jnp.where(iota == ck, 1e30, d)
    wk = [1.0 / (v + 1e-8) for v in vks]
    wsum = wk[0] + wk[1] + wk[2]
    Wm = jnp.zeros((Tn, S), jnp.float32)
    for k in range(3):
        Wm = Wm + jnp.where(iota == iks[k], wk[k] / wsum, 0.0)
    interp = _dot(Wm, p2)

    h = jnp.concatenate([p1_ref[0], interp], axis=1) if has_p1 else interp
    for (W, b) in mlp:
        h = _relu(_dot(h, W) + b)
    out_ref[0] = h


def _fp_call(x1_rows, x2_cols, p2_rows, p1_rows, p, Tn):
    B, N, _ = x1_rows.shape
    S = x2_cols.shape[2]
    C2 = p2_rows.shape[2]
    mlp = [_fold_bn(L) for L in p["mlp"]]
    wargs = []
    for W, b in mlp:
        wargs += [W, b]
    Cout = mlp[-1][0].shape[1]
    grid = (B, N // Tn)
    in_specs = [
        pl.BlockSpec((1, Tn, 3), lambda b_, t_: (b_, t_, 0)),
        pl.BlockSpec((1, 3, S), lambda b_, t_: (b_, 0, 0)),
        pl.BlockSpec((1, S, C2), lambda b_, t_: (b_, 0, 0)),
    ]
    args = [x1_rows, x2_cols, p2_rows]
    if p1_rows is not None:
        C1 = p1_rows.shape[2]
        in_specs.append(pl.BlockSpec((1, Tn, C1), lambda b_, t_: (b_, t_, 0)))
        args.append(p1_rows)
    in_specs += [pl.BlockSpec(w.shape, lambda b_, t_: (0,) * w.ndim)
                 for w in wargs]
    args += wargs
    return pl.pallas_call(
        functools.partial(_fp_kernel, n_mlp=len(mlp),
                          has_p1=p1_rows is not None),
        grid=grid,
        in_specs=in_specs,
        out_specs=pl.BlockSpec((1, Tn, Cout), lambda b_, t_: (b_, t_, 0)),
        out_shape=jax.ShapeDtypeStruct((B, N, Cout), jnp.float32),
    )(*args)


def _head_kernel(x_ref, w1_ref, b1_ref, w2_ref, b2_ref, out_ref):
    h = _relu(_dot(x_ref[...], w1_ref[...]) + b1_ref[...])
    y = _dot(h, w2_ref[...]) + b2_ref[...]
    mx = jnp.max(y, axis=1, keepdims=True)
    lse = jnp.log(jnp.sum(jnp.exp(y - mx), axis=1, keepdims=True)) + mx
    out_ref[...] = y - lse


def _head_call(rows, p1, p2, Tr):
    R, C = rows.shape
    W1, b1 = _fold_bn(p1)
    W2, b2 = p2["W"], p2["b"].reshape(1, -1)
    NC = W2.shape[1]
    grid = (R // Tr,)
    return pl.pallas_call(
        _head_kernel,
        grid=grid,
        in_specs=[
            pl.BlockSpec((Tr, C), lambda t_: (t_, 0)),
            pl.BlockSpec(W1.shape, lambda t_: (0, 0)),
            pl.BlockSpec(b1.shape, lambda t_: (0, 0)),
            pl.BlockSpec(W2.shape, lambda t_: (0, 0)),
            pl.BlockSpec(b2.shape, lambda t_: (0, 0)),
        ],
        out_specs=pl.BlockSpec((Tr, NC), lambda t_: (t_, 0)),
        out_shape=jax.ShapeDtypeStruct((R, NC), jnp.float32),
    )(rows, W1, b1, W2, b2)


_LEVELS = [
    (1024, 0.1, 256),
    (256, 0.2, 256),
    (64, 0.4, 64),
    (16, 0.8, 16),
]


def kernel(xyz, params):
    B = xyz.shape[0]
    xyz_cols = xyz[:, 0:3, :]
    xyz_rows = xyz_cols.transpose(0, 2, 1)
    pts_rows = xyz[:, 3:6, :].transpose(0, 2, 1)

    new_all = _fps_call(xyz_cols, tuple(c[0] for c in _LEVELS))

    xs_cols = [xyz_cols]
    xs_rows = [xyz_rows]
    ps_rows = [pts_rows]
    for li, (npoint, radius, Ts) in enumerate(_LEVELS):
        p = params[f"ra{li + 1}"]
        new_cols = new_all[li]
        new_rows = new_cols.transpose(0, 2, 1)
        tbl = jnp.concatenate([xs_rows[-1], ps_rows[-1]], axis=2)
        out = _sa_call(tbl, xs_cols[-1], new_rows, p, radius, 16, Ts)
        xs_cols.append(new_cols)
        xs_rows.append(new_rows)
        ps_rows.append(out)

    fp_cfg = [
        ("fp4", 3, 4, 64),
        ("fp3", 2, 3, 256),
        ("fp2", 1, 2, 512),
        ("fp1", 0, 1, 1024),
    ]
    cur = ps_rows[4]
    for name, i1, i2, Tn in fp_cfg:
        p = params[name]
        p1 = ps_rows[i1] if i1 > 0 else None
        cur = _fp_call(xs_rows[i1], xs_cols[i2], cur, p1, p, Tn)

    l0_rows = cur
    N = l0_rows.shape[1]
    rows = l0_rows.reshape(B * N, l0_rows.shape[2])
    x = _head_call(rows, params["head1"], params["head2"], 2048)
    x = x.reshape(B, N, x.shape[1])
    l0_out = l0_rows.transpose(0, 2, 1)
    return x, l0_out


# speedup vs baseline: 13.8651x; 1.0339x over previous
import functools

import jax
import jax.numpy as jnp
from jax.experimental import pallas as pl


def _relu(x):
    return jnp.maximum(x, 0.0)


def _dot(a, b):
    return jax.lax.dot_general(a, b, (((1,), (0,)), ((), ())),
                               preferred_element_type=jnp.float32)


def _fold_bn(layer):
    W, b = layer["W"], layer["b"]
    if "gamma" in layer:
        g, beta = layer["gamma"], layer["beta"]
        W = W * g[None, :]
        b = b * g + beta
    return W, b.reshape(1, -1)


def _fps_stage(x, y, z, npoint):
    B, N = x.shape
    iota_n = jax.lax.broadcasted_iota(jnp.int32, (B, N), 1)
    iota_p = jax.lax.broadcasted_iota(jnp.int32, (B, npoint), 1)

    def body(i, st):
        dist, far, ax, ay, az = st
        oh = (iota_n == far).astype(jnp.float32)
        cx = jnp.sum(oh * x, axis=1, keepdims=True)
        cy = jnp.sum(oh * y, axis=1, keepdims=True)
        cz = jnp.sum(oh * z, axis=1, keepdims=True)
        d = (x - cx) ** 2 + (y - cy) ** 2 + (z - cz) ** 2
        dist = jnp.minimum(dist, d)
        mx = jnp.max(dist, axis=1, keepdims=True)
        far = jnp.min(jnp.where(dist == mx, iota_n, N), axis=1, keepdims=True)
        sel = iota_p == i
        ax = jnp.where(sel, cx, ax)
        ay = jnp.where(sel, cy, ay)
        az = jnp.where(sel, cz, az)
        return dist, far, ax, ay, az

    init = (jnp.full((B, N), 1e10, jnp.float32),
            jnp.zeros((B, 1), jnp.int32),
            jnp.zeros((B, npoint), jnp.float32),
            jnp.zeros((B, npoint), jnp.float32),
            jnp.zeros((B, npoint), jnp.float32))
    _, _, ax, ay, az = jax.lax.fori_loop(0, npoint, body, init)
    return ax, ay, az


def _fps_kernel(xyz_ref, o1_ref, o2_ref, o3_ref, o4_ref, *, npoints):
    x = xyz_ref[:, 0, :]
    y = xyz_ref[:, 1, :]
    z = xyz_ref[:, 2, :]
    for npoint, o_ref in zip(npoints, (o1_ref, o2_ref, o3_ref, o4_ref)):
        x, y, z = _fps_stage(x, y, z, npoint)
        o_ref[:, 0, :] = x
        o_ref[:, 1, :] = y
        o_ref[:, 2, :] = z


def _fps_call(xyz_cols, npoints):
    B = xyz_cols.shape[0]
    return pl.pallas_call(
        functools.partial(_fps_kernel, npoints=npoints),
        out_shape=[jax.ShapeDtypeStruct((B, 3, n), jnp.float32)
                   for n in npoints],
    )(xyz_cols)


def _sa_kernel(x1_ref, tblT_ref, newc_ref, tri_ref, *w_refs, r2, K, n_mlp):
    out_ref = w_refs[-1]
    w_refs = w_refs[:-1]
    ws = [w_refs[i][...] for i in range(len(w_refs))]
    mlp = [(ws[2 * i], ws[2 * i + 1]) for i in range(n_mlp)]
    o = 2 * n_mlp
    lfa1W, lfa1b = ws[o], ws[o + 1]
    attW, attb = ws[o + 2], ws[o + 3]
    lfa2W, lfa2b = ws[o + 4], ws[o + 5]

    x1 = x1_ref[0]
    tblT = tblT_ref[0]
    newc = newc_ref[0]
    D, N = tblT.shape
    Ts = newc.shape[1]

    sqx = jnp.sum(x1 * x1, axis=1, keepdims=True)
    sqn = jnp.sum(newc * newc, axis=0, keepdims=True)
    d2 = sqn + sqx - 2.0 * _dot(x1, newc)

    m = d2 <= r2
    rank = _dot(tri_ref[...], m.astype(jnp.bfloat16))
    cnt = rank[N - 1:N, :]
    rankm = jnp.where(m, rank, 0.0)

    sub = jnp.concatenate(
        [newc, jnp.zeros((D - 3, Ts), jnp.float32)], axis=0)

    hA = None
    gs = []
    aa = []
    f0 = None
    for k in range(K):
        oh = (rankm == float(k + 1)).astype(jnp.float32)
        f = _dot(tblT, oh)
        if k == 0:
            f0 = f
        else:
            f = jnp.where(cnt >= float(k + 1), f, f0)
        f = f - sub
        h = f
        for (W, b) in mlp:
            h = _relu(_dot(W, h) + b)
        hA = h if hA is None else jnp.maximum(hA, h)
        g = _relu(_dot(lfa1W, f) + lfa1b)
        gs.append(g)
        aa.append(_dot(attW, g) + attb)

    mx = aa[0]
    for a in aa[1:]:
        mx = jnp.maximum(mx, a)
    es = [jnp.exp(a - mx) for a in aa]
    ssum = es[0]
    for e in es[1:]:
        ssum = ssum + e
    pooled = gs[0] * es[0]
    for g, e in zip(gs[1:], es[1:]):
        pooled = pooled + g * e
    pooled = pooled / ssum
    bB = _relu(_dot(lfa2W, pooled) + lfa2b)

    out_ref[0] = jnp.concatenate([hA, bB], axis=0)


def _sa_call(x1_rows, tblT, new_cols, p, radius, K, Ts):
    B, D, N = tblT.shape
    S = new_cols.shape[2]

    def tw(Wb):
        W, b = Wb
        return W.T, b.reshape(-1, 1)

    mlp = [tw(_fold_bn(L)) for L in p["mlp"]]
    lfa1 = tw(_fold_bn(p["lfa1"]))
    att = (p["att"]["W"].T, p["att"]["b"].reshape(-1, 1))
    lfa2 = tw(_fold_bn(p["lfa2"]))
    wargs = []
    for W, b in mlp + [lfa1, att, lfa2]:
        wargs += [W, b]
    Cout = mlp[-1][0].shape[0] + lfa2[0].shape[0]
    grid = (B, S // Ts)
    tri = jnp.tril(jnp.ones((N, N), jnp.bfloat16))
    wspecs = [pl.BlockSpec(w.shape, lambda b_, t_: (0,) * w.ndim)
              for w in wargs]
    return pl.pallas_call(
        functools.partial(_sa_kernel, r2=radius * radius, K=K,
                          n_mlp=len(mlp)),
        grid=grid,
        in_specs=[
            pl.BlockSpec((1, N, 3), lambda b_, t_: (b_, 0, 0)),
            pl.BlockSpec((1, D, N), lambda b_, t_: (b_, 0, 0)),
            pl.BlockSpec((1, 3, Ts), lambda b_, t_: (b_, 0, t_)),
            pl.BlockSpec((N, N), lambda b_, t_: (0, 0)),
        ] + wspecs,
        out_specs=pl.BlockSpec((1, Cout, Ts), lambda b_, t_: (b_, 0, t_)),
        out_shape=jax.ShapeDtypeStruct((B, Cout, S), jnp.float32),
    )(x1_rows, tblT, new_cols, tri, *wargs)


def _fp_kernel(*refs, n_mlp, has_p1):
    if has_p1:
        x1_ref, x2c_ref, p2_ref, p1_ref = refs[:4]
        w_refs = refs[4:-1]
    else:
        x1_ref, x2c_ref, p2_ref = refs[:3]
        w_refs = refs[3:-1]
    out_ref = refs[-1]
    ws = [w_refs[i][...] for i in range(len(w_refs))]
    mlp = [(ws[2 * i], ws[2 * i + 1]) for i in range(n_mlp)]

    x1 = x1_ref[0]
    x2c = x2c_ref[0]
    p2 = p2_ref[0]
    Tn = x1.shape[0]
    S = x2c.shape[1]

    sq1 = jnp.sum(x1 * x1, axis=1, keepdims=True)
    sq2 = jnp.sum(x2c * x2c, axis=0, keepdims=True)
    d2 = sq1 + sq2 - 2.0 * _dot(x1, x2c)
    iota = jax.lax.broadcasted_iota(jnp.int32, (Tn, S), 1)

    d = d2
    iks = []
    vks = []
    for _ in range(3):
        mn = jnp.min(d, axis=1, keepdims=True)
        ck = jnp.min(jnp.where(d == mn, iota, S), axis=1, keepdims=True)
        iks.append(ck)
        vks.append(mn)
        d = jnp.where(iota == ck, 1e30, d)
    wk = [1.0 / (v + 1e-8) for v in vks]
    wsum = wk[0] + wk[1] + wk[2]
    Wm = jnp.zeros((Tn, S), jnp.float32)
    for k in range(3):
        Wm = Wm + jnp.where(iota == iks[k], wk[k] / wsum, 0.0)
    interp = _dot(Wm, p2)

    h = jnp.concatenate([p1_ref[0], interp], axis=1) if has_p1 else interp
    for (W, b) in mlp:
        h = _relu(_dot(h, W) + b)
    out_ref[0] = h


def _fp_call(x1_rows, x2_cols, p2_rows, p1_rows, p, Tn):
    B, N, _ = x1_rows.shape
    S = x2_cols.shape[2]
    C2 = p2_rows.shape[2]
    mlp = [_fold_bn(L) for L in p["mlp"]]
    wargs = []
    for W, b in mlp:
        wargs += [W, b]
    Cout = mlp[-1][0].shape[1]
    grid = (B, N // Tn)
    in_specs = [
        pl.BlockSpec((1, Tn, 3), lambda b_, t_: (b_, t_, 0)),
        pl.BlockSpec((1, 3, S), lambda b_, t_: (b_, 0, 0)),
        pl.BlockSpec((1, S, C2), lambda b_, t_: (b_, 0, 0)),
    ]
    args = [x1_rows, x2_cols, p2_rows]
    if p1_rows is not None:
        C1 = p1_rows.shape[2]
        in_specs.append(pl.BlockSpec((1, Tn, C1), lambda b_, t_: (b_, t_, 0)))
        args.append(p1_rows)
    in_specs += [pl.BlockSpec(w.shape, lambda b_, t_: (0,) * w.ndim)
                 for w in wargs]
    args += wargs
    return pl.pallas_call(
        functools.partial(_fp_kernel, n_mlp=len(mlp),
                          has_p1=p1_rows is not None),
        grid=grid,
        in_specs=in_specs,
        out_specs=pl.BlockSpec((1, Tn, Cout), lambda b_, t_: (b_, t_, 0)),
        out_shape=jax.ShapeDtypeStruct((B, N, Cout), jnp.float32),
    )(*args)


def _head_kernel(x_ref, w1_ref, b1_ref, w2_ref, b2_ref, out_ref):
    h = _relu(_dot(x_ref[...], w1_ref[...]) + b1_ref[...])
    y = _dot(h, w2_ref[...]) + b2_ref[...]
    mx = jnp.max(y, axis=1, keepdims=True)
    lse = jnp.log(jnp.sum(jnp.exp(y - mx), axis=1, keepdims=True)) + mx
    out_ref[...] = y - lse


def _head_call(rows, p1, p2, Tr):
    R, C = rows.shape
    W1, b1 = _fold_bn(p1)
    W2, b2 = p2["W"], p2["b"].reshape(1, -1)
    NC = W2.shape[1]
    grid = (R // Tr,)
    return pl.pallas_call(
        _head_kernel,
        grid=grid,
        in_specs=[
            pl.BlockSpec((Tr, C), lambda t_: (t_, 0)),
            pl.BlockSpec(W1.shape, lambda t_: (0, 0)),
            pl.BlockSpec(b1.shape, lambda t_: (0, 0)),
            pl.BlockSpec(W2.shape, lambda t_: (0, 0)),
            pl.BlockSpec(b2.shape, lambda t_: (0, 0)),
        ],
        out_specs=pl.BlockSpec((Tr, NC), lambda t_: (t_, 0)),
        out_shape=jax.ShapeDtypeStruct((R, NC), jnp.float32),
    )(rows, W1, b1, W2, b2)


_LEVELS = [
    (1024, 0.1, 256),
    (256, 0.2, 256),
    (64, 0.4, 64),
    (16, 0.8, 16),
]


def kernel(xyz, params):
    B = xyz.shape[0]
    xyz_cols = xyz[:, 0:3, :]
    xyz_rows = xyz_cols.transpose(0, 2, 1)
    pts_rows = xyz[:, 3:6, :].transpose(0, 2, 1)

    new_all = _fps_call(xyz_cols, tuple(c[0] for c in _LEVELS))

    xs_cols = [xyz_cols]
    xs_rows = [xyz_rows]
    ps_cols = [xyz[:, 3:6, :]]
    for li, (npoint, radius, Ts) in enumerate(_LEVELS):
        p = params[f"ra{li + 1}"]
        new_cols = new_all[li]
        tblT = jnp.concatenate([xs_cols[-1], ps_cols[-1]], axis=1)
        out = _sa_call(xs_rows[-1], tblT, new_cols, p, radius, 16, Ts)
        xs_cols.append(new_cols)
        xs_rows.append(new_cols.transpose(0, 2, 1))
        ps_cols.append(out)

    fp_cfg = [
        ("fp4", 3, 4, 64),
        ("fp3", 2, 3, 256),
        ("fp2", 1, 2, 512),
        ("fp1", 0, 1, 1024),
    ]
    cur = ps_cols[4].transpose(0, 2, 1)
    for name, i1, i2, Tn in fp_cfg:
        p = params[name]
        p1 = ps_cols[i1].transpose(0, 2, 1) if i1 > 0 else None
        cur = _fp_call(xs_rows[i1], xs_cols[i2], cur, p1, p, Tn)

    l0_rows = cur
    N = l0_rows.shape[1]
    rows = l0_rows.reshape(B * N, l0_rows.shape[2])
    x = _head_call(rows, params["head1"], params["head2"], 2048)
    x = x.reshape(B, N, x.shape[1])
    l0_out = l0_rows.transpose(0, 2, 1)
    return x, l0_out


# speedup vs baseline: 14.7695x; 1.0652x over previous
import functools

import jax
import jax.numpy as jnp
from jax.experimental import pallas as pl


def _relu(x):
    return jnp.maximum(x, 0.0)


def _dot(a, b):
    return jax.lax.dot_general(a, b, (((1,), (0,)), ((), ())),
                               preferred_element_type=jnp.float32)


def _doth(a, b):
    return jax.lax.dot_general(a, b, (((1,), (0,)), ((), ())),
                               preferred_element_type=jnp.float32,
                               precision=jax.lax.Precision.HIGHEST)


def _dotb(a, b):
    return jax.lax.dot_general(a, b, (((1,), (0,)), ((), ())),
                               preferred_element_type=jnp.float32)


def _prefix_count(m, N, Ts):
    CH = min(N, 128)
    nch = N // CH
    ri = jax.lax.broadcasted_iota(jnp.int32, (CH, CH), 0)
    ci = jax.lax.broadcasted_iota(jnp.int32, (CH, CH), 1)
    tri = (ri >= ci).astype(jnp.bfloat16)
    mb = m.astype(jnp.bfloat16)
    intra = [_dotb(tri, mb[c * CH:(c + 1) * CH, :]) for c in range(nch)]
    if nch == 1:
        rank = intra[0]
        cnt = rank[CH - 1:CH, :]
        return rank, cnt
    tot = jnp.concatenate([ic[CH - 1:CH, :] for ic in intra], axis=0)
    ri2 = jax.lax.broadcasted_iota(jnp.int32, (nch, nch), 0)
    ci2 = jax.lax.broadcasted_iota(jnp.int32, (nch, nch), 1)
    tri2 = (ri2 > ci2).astype(jnp.bfloat16)
    off = _dotb(tri2, tot.astype(jnp.bfloat16))
    rank = jnp.concatenate(
        [intra[c] + off[c:c + 1, :] for c in range(nch)], axis=0)
    cnt = jnp.sum(tot, axis=0, keepdims=True)
    return rank, cnt


def _fold_bn(layer):
    W, b = layer["W"], layer["b"]
    if "gamma" in layer:
        g, beta = layer["gamma"], layer["beta"]
        W = W * g[None, :]
        b = b * g + beta
    return W, b.reshape(1, -1)


def _fps_stage(x, y, z, npoint):
    B, N = x.shape
    iota_n = jax.lax.broadcasted_iota(jnp.int32, (B, N), 1)
    iota_p = jax.lax.broadcasted_iota(jnp.int32, (B, npoint), 1)

    def body(i, st):
        dist, far, ax, ay, az = st
        oh = (iota_n == far).astype(jnp.float32)
        cx = jnp.sum(oh * x, axis=1, keepdims=True)
        cy = jnp.sum(oh * y, axis=1, keepdims=True)
        cz = jnp.sum(oh * z, axis=1, keepdims=True)
        d = (x - cx) ** 2 + (y - cy) ** 2 + (z - cz) ** 2
        dist = jnp.minimum(dist, d)
        mx = jnp.max(dist, axis=1, keepdims=True)
        far = jnp.min(jnp.where(dist == mx, iota_n, N), axis=1, keepdims=True)
        sel = iota_p == i
        ax = jnp.where(sel, cx, ax)
        ay = jnp.where(sel, cy, ay)
        az = jnp.where(sel, cz, az)
        return dist, far, ax, ay, az

    init = (jnp.full((B, N), 1e10, jnp.float32),
            jnp.zeros((B, 1), jnp.int32),
            jnp.zeros((B, npoint), jnp.float32),
            jnp.zeros((B, npoint), jnp.float32),
            jnp.zeros((B, npoint), jnp.float32))
    _, _, ax, ay, az = jax.lax.fori_loop(0, npoint, body, init)
    return ax, ay, az


def _fps_kernel(xyz_ref, o1_ref, o2_ref, o3_ref, o4_ref, *, npoints):
    x = xyz_ref[:, 0, :]
    y = xyz_ref[:, 1, :]
    z = xyz_ref[:, 2, :]
    for npoint, o_ref in zip(npoints, (o1_ref, o2_ref, o3_ref, o4_ref)):
        x, y, z = _fps_stage(x, y, z, npoint)
        o_ref[:, 0, :] = x
        o_ref[:, 1, :] = y
        o_ref[:, 2, :] = z


def _fps_call(xyz_cols, npoints):
    B = xyz_cols.shape[0]
    return pl.pallas_call(
        functools.partial(_fps_kernel, npoints=npoints),
        out_shape=[jax.ShapeDtypeStruct((B, 3, n), jnp.float32)
                   for n in npoints],
    )(xyz_cols)


def _sa_kernel(x1_ref, tblT_ref, newc_ref, *w_refs, r2, K, n_mlp):
    out_ref = w_refs[-1]
    w_refs = w_refs[:-1]
    ws = [w_refs[i][...] for i in range(len(w_refs))]
    mlp = [(ws[2 * i], ws[2 * i + 1]) for i in range(n_mlp)]
    o = 2 * n_mlp
    lfa1W, lfa1b = ws[o], ws[o + 1]
    attW, attb = ws[o + 2], ws[o + 3]
    lfa2W, lfa2b = ws[o + 4], ws[o + 5]

    x1 = x1_ref[0]
    tblT = tblT_ref[0]
    newc = newc_ref[0]
    D, N = tblT.shape
    Ts = newc.shape[1]

    sqx = jnp.sum(x1 * x1, axis=1, keepdims=True)
    sqn = jnp.sum(newc * newc, axis=0, keepdims=True)
    d2 = sqn + sqx - 2.0 * _doth(x1, newc)

    m = d2 <= r2
    rank, cnt = _prefix_count(m, N, Ts)
    rankm = jnp.where(m, rank, 0.0)

    sub = jnp.concatenate(
        [newc, jnp.zeros((D - 3, Ts), jnp.float32)], axis=0)

    hA = None
    gs = []
    aa = []
    f0 = None
    for k in range(K):
        oh = (rankm == float(k + 1)).astype(jnp.float32)
        f = _dot(tblT, oh)
        if k == 0:
            f0 = f
        else:
            f = jnp.where(cnt >= float(k + 1), f, f0)
        f = f - sub
        h = f
        for (W, b) in mlp:
            h = _relu(_dot(W, h) + b)
        hA = h if hA is None else jnp.maximum(hA, h)
        g = _relu(_dot(lfa1W, f) + lfa1b)
        gs.append(g)
        aa.append(_dot(attW, g) + attb)

    mx = aa[0]
    for a in aa[1:]:
        mx = jnp.maximum(mx, a)
    es = [jnp.exp(a - mx) for a in aa]
    ssum = es[0]
    for e in es[1:]:
        ssum = ssum + e
    pooled = gs[0] * es[0]
    for g, e in zip(gs[1:], es[1:]):
        pooled = pooled + g * e
    pooled = pooled / ssum
    bB = _relu(_dot(lfa2W, pooled) + lfa2b)

    out_ref[0] = jnp.concatenate([hA, bB], axis=0)


def _sa_call(x1_rows, tblT, new_cols, p, radius, K, Ts):
    B, D, N = tblT.shape
    S = new_cols.shape[2]

    def tw(Wb):
        W, b = Wb
        return W.T, b.reshape(-1, 1)

    mlp = [tw(_fold_bn(L)) for L in p["mlp"]]
    lfa1 = tw(_fold_bn(p["lfa1"]))
    att = (p["att"]["W"].T, p["att"]["b"].reshape(-1, 1))
    lfa2 = tw(_fold_bn(p["lfa2"]))
    wargs = []
    for W, b in mlp + [lfa1, att, lfa2]:
        wargs += [W, b]
    Cout = mlp[-1][0].shape[0] + lfa2[0].shape[0]
    grid = (B, S // Ts)
    wspecs = [pl.BlockSpec(w.shape, lambda b_, t_: (0,) * w.ndim)
              for w in wargs]
    return pl.pallas_call(
        functools.partial(_sa_kernel, r2=radius * radius, K=K,
                          n_mlp=len(mlp)),
        grid=grid,
        in_specs=[
            pl.BlockSpec((1, N, 3), lambda b_, t_: (b_, 0, 0)),
            pl.BlockSpec((1, D, N), lambda b_, t_: (b_, 0, 0)),
            pl.BlockSpec((1, 3, Ts), lambda b_, t_: (b_, 0, t_)),
        ] + wspecs,
        out_specs=pl.BlockSpec((1, Cout, Ts), lambda b_, t_: (b_, 0, t_)),
        out_shape=jax.ShapeDtypeStruct((B, Cout, S), jnp.float32),
    )(x1_rows, tblT, new_cols, *wargs)


def _fp_kernel(*refs, n_mlp, has_p1):
    if has_p1:
        x1_ref, x2c_ref, p2_ref, p1_ref = refs[:4]
        w_refs = refs[4:-1]
    else:
        x1_ref, x2c_ref, p2_ref = refs[:3]
        w_refs = refs[3:-1]
    out_ref = refs[-1]
    ws = [w_refs[i][...] for i in range(len(w_refs))]
    mlp = [(ws[2 * i], ws[2 * i + 1]) for i in range(n_mlp)]

    x1 = x1_ref[0]
    x2c = x2c_ref[0]
    p2 = p2_ref[0]
    Tn = x1.shape[0]
    S = x2c.shape[1]

    sq1 = jnp.sum(x1 * x1, axis=1, keepdims=True)
    sq2 = jnp.sum(x2c * x2c, axis=0, keepdims=True)
    d2 = sq1 + sq2 - 2.0 * _doth(x1, x2c)
    iota = jax.lax.broadcasted_iota(jnp.int32, (Tn, S), 1)

    d = d2
    iks = []
    vks = []
    for _ in range(3):
        mn = jnp.min(d, axis=1, keepdims=True)
        ck = jnp.min(jnp.where(d == mn, iota, S), axis=1, keepdims=True)
        iks.append(ck)
        vks.append(mn)
        d = jnp.where(iota == ck, 1e30, d)
    wk = [1.0 / (v + 1e-8) for v in vks]
    wsum = wk[0] + wk[1] + wk[2]
    Wm = jnp.zeros((Tn, S), jnp.float32)
    for k in range(3):
        Wm = Wm + jnp.where(iota == iks[k], wk[k] / wsum, 0.0)
    interp = _dot(Wm, p2)

    h = jnp.concatenate([p1_ref[0], interp], axis=1) if has_p1 else interp
    for (W, b) in mlp:
        h = _relu(_dot(h, W) + b)
    out_ref[0] = h


def _fp_call(x1_rows, x2_cols, p2_rows, p1_rows, p, Tn):
    B, N, _ = x1_rows.shape
    S = x2_cols.shape[2]
    C2 = p2_rows.shape[2]
    mlp = [_fold_bn(L) for L in p["mlp"]]
    wargs = []
    for W, b in mlp:
        wargs += [W, b]
    Cout = mlp[-1][0].shape[1]
    grid = (B, N // Tn)
    in_specs = [
        pl.BlockSpec((1, Tn, 3), lambda b_, t_: (b_, t_, 0)),
        pl.BlockSpec((1, 3, S), lambda b_, t_: (b_, 0, 0)),
        pl.BlockSpec((1, S, C2), lambda b_, t_: (b_, 0, 0)),
    ]
    args = [x1_rows, x2_cols, p2_rows]
    if p1_rows is not None:
        C1 = p1_rows.shape[2]
        in_specs.append(pl.BlockSpec((1, Tn, C1), lambda b_, t_: (b_, t_, 0)))
        args.append(p1_rows)
    in_specs += [pl.BlockSpec(w.shape, lambda b_, t_: (0,) * w.ndim)
                 for w in wargs]
    args += wargs
    return pl.pallas_call(
        functools.partial(_fp_kernel, n_mlp=len(mlp),
                          has_p1=p1_rows is not None),
        grid=grid,
        in_specs=in_specs,
        out_specs=pl.BlockSpec((1, Tn, Cout), lambda b_, t_: (b_, t_, 0)),
        out_shape=jax.ShapeDtypeStruct((B, N, Cout), jnp.float32),
    )(*args)


def _head_kernel(x_ref, w1_ref, b1_ref, w2_ref, b2_ref, out_ref):
    h = _relu(_dot(x_ref[...], w1_ref[...]) + b1_ref[...])
    y = _dot(h, w2_ref[...]) + b2_ref[...]
    mx = jnp.max(y, axis=1, keepdims=True)
    lse = jnp.log(jnp.sum(jnp.exp(y - mx), axis=1, keepdims=True)) + mx
    out_ref[...] = y - lse


def _head_call(rows, p1, p2, Tr):
    R, C = rows.shape
    W1, b1 = _fold_bn(p1)
    W2, b2 = p2["W"], p2["b"].reshape(1, -1)
    NC = W2.shape[1]
    grid = (R // Tr,)
    return pl.pallas_call(
        _head_kernel,
        grid=grid,
        in_specs=[
            pl.BlockSpec((Tr, C), lambda t_: (t_, 0)),
            pl.BlockSpec(W1.shape, lambda t_: (0, 0)),
            pl.BlockSpec(b1.shape, lambda t_: (0, 0)),
            pl.BlockSpec(W2.shape, lambda t_: (0, 0)),
            pl.BlockSpec(b2.shape, lambda t_: (0, 0)),
        ],
        out_specs=pl.BlockSpec((Tr, NC), lambda t_: (t_, 0)),
        out_shape=jax.ShapeDtypeStruct((R, NC), jnp.float32),
    )(rows, W1, b1, W2, b2)


_LEVELS = [
    (1024, 0.1, 256),
    (256, 0.2, 256),
    (64, 0.4, 64),
    (16, 0.8, 16),
]


def kernel(xyz, params):
    B = xyz.shape[0]
    xyz_cols = xyz[:, 0:3, :]
    xyz_rows = xyz_cols.transpose(0, 2, 1)
    pts_rows = xyz[:, 3:6, :].transpose(0, 2, 1)

    new_all = _fps_call(xyz_cols, tuple(c[0] for c in _LEVELS))

    xs_cols = [xyz_cols]
    xs_rows = [xyz_rows]
    ps_cols = [xyz[:, 3:6, :]]
    for li, (npoint, radius, Ts) in enumerate(_LEVELS):
        p = params[f"ra{li + 1}"]
        new_cols = new_all[li]
        tblT = jnp.concatenate([xs_cols[-1], ps_cols[-1]], axis=1)
        out = _sa_call(xs_rows[-1], tblT, new_cols, p, radius, 16, Ts)
        xs_cols.append(new_cols)
        xs_rows.append(new_cols.transpose(0, 2, 1))
        ps_cols.append(out)

    fp_cfg = [
        ("fp4", 3, 4, 64),
        ("fp3", 2, 3, 256),
        ("fp2", 1, 2, 512),
        ("fp1", 0, 1, 1024),
    ]
    cur = ps_cols[4].transpose(0, 2, 1)
    for name, i1, i2, Tn in fp_cfg:
        p = params[name]
        p1 = ps_cols[i1].transpose(0, 2, 1) if i1 > 0 else None
        cur = _fp_call(xs_rows[i1], xs_cols[i2], cur, p1, p, Tn)

    l0_rows = cur
    N = l0_rows.shape[1]
    rows = l0_rows.reshape(B * N, l0_rows.shape[2])
    x = _head_call(rows, params["head1"], params["head2"], 2048)
    x = x.reshape(B, N, x.shape[1])
    l0_out = l0_rows.transpose(0, 2, 1)
    return x, l0_out
